# Initial kernel scaffold; baseline (speedup 1.0000x reference)
import jax
import jax.numpy as jnp
from jax.experimental import pallas as pl


def kernel(node_features, edge_features, node_attn_weights, edge_attn_weights, Wn, bn, We, be, edge_index):
    raise NotImplementedError("write your pallas kernel here")


# speedup vs baseline: 1.0032x; 1.0032x over previous
import functools

import jax
import jax.numpy as jnp
from jax import lax
from jax.experimental import pallas as pl
from jax.experimental.pallas import tpu as pltpu
from jax.experimental.pallas import tpu_sc as plsc

_N = 100000
_E = 1600000
_H = 8
_NR = 20
_CHE = _NR * 128
_G = _E // _CHE
_NW = 32


def _sc_agg_body(naw_ref, dst_ref, eas_ref, part_ref, maxp_ref,
                 w_buf, d_buf, v_buf, m_buf, acc):
    c = lax.axis_index("c")
    s = lax.axis_index("s")
    w = s * 2 + c

    def _zb(i, carry):
        acc[pl.ds(i * 16, 16)] = jnp.zeros((16,), jnp.float32)
        return carry
    lax.fori_loop(0, _N // 16, _zb, 0)

    lanes8 = lax.iota(jnp.int32, 16) * 8
    nj = (_G - w + _NW - 1) // _NW

    def _chunk(j, m):
        g = w + j * _NW
        e0 = g * _CHE
        pltpu.sync_copy(naw_ref.at[pl.ds(e0 * _H, _CHE * _H)], w_buf)
        pltpu.sync_copy(dst_ref.at[pl.ds(e0, _CHE)], d_buf)

        def _grp(i, m):
            off = i * 128
            acc_v = plsc.load_gather(w_buf, [lanes8 + off])
            for h in range(1, _H):
                acc_v = acc_v + plsc.load_gather(w_buf, [lanes8 + (off + h)])
            mean = acc_v * jnp.float32(1.0 / _H)
            v_buf[pl.ds(i * 16, 16)] = mean
            idx = d_buf[pl.ds(i * 16, 16)]
            plsc.addupdate_scatter(acc, [idx], mean)
            return jnp.maximum(m, mean)

        m = lax.fori_loop(0, _CHE // 16, _grp, m)
        pltpu.sync_copy(v_buf, eas_ref.at[pl.ds(e0, _CHE)])
        return m

    m = lax.fori_loop(0, nj, _chunk, jnp.zeros((16,), jnp.float32))
    m_buf[...] = m
    pltpu.sync_copy(m_buf, maxp_ref.at[pl.ds(w * 16, 16)])
    pltpu.sync_copy(acc, part_ref.at[w])


_sc_agg = pl.kernel(
    _sc_agg_body,
    out_type=[
        jax.ShapeDtypeStruct((_E,), jnp.float32),
        jax.ShapeDtypeStruct((_NW, _N), jnp.float32),
        jax.ShapeDtypeStruct((_NW * 16,), jnp.float32),
    ],
    mesh=plsc.VectorSubcoreMesh(core_axis_name="c", subcore_axis_name="s"),
    compiler_params=pltpu.CompilerParams(needs_layout_passes=False),
    scratch_types=[
        pltpu.VMEM((_CHE * _H,), jnp.float32),
        pltpu.VMEM((_CHE,), jnp.int32),
        pltpu.VMEM((_CHE,), jnp.float32),
        pltpu.VMEM((16,), jnp.float32),
        pltpu.VMEM((_N,), jnp.float32),
    ],
)


_BN = 2000
_NBN = _N // _BN
_BE = 2560
_NBE = _E // _BE

def _tcn1_body(nf_ref, part_ref, wnc_ref, base_ref, agg_ref, mx_ref):
    i = pl.program_id(0)
    blk = nf_ref[...].astype(jnp.bfloat16)
    wnt = jnp.transpose(wnc_ref[...]).astype(jnp.bfloat16)
    base = lax.dot_general(wnt, blk, (((1,), (1,)), ((), ())),
                           preferred_element_type=jnp.float32)
    base_ref[0] = base
    agg = jnp.sum(part_ref[...], axis=0)
    agg_ref[...] = agg
    m = jnp.max(agg)

    @pl.when(i == 0)
    def _():
        mx_ref[0, 0] = m

    @pl.when(i > 0)
    def _():
        mx_ref[0, 0] = jnp.maximum(mx_ref[0, 0], m)


def _tcn1(nf, part4, wnc):
    return pl.pallas_call(
        _tcn1_body,
        grid=(_NBN,),
        in_specs=[
            pl.BlockSpec((_BN, 128), lambda i: (i, 0)),
            pl.BlockSpec((_NW, 1, 1, _BN), lambda i: (0, i, 0, 0)),
            pl.BlockSpec((128, 1), lambda i: (0, 0)),
        ],
        out_specs=[
            pl.BlockSpec((1, 1, _BN), lambda i: (i, 0, 0)),
            pl.BlockSpec((1, 1, _BN), lambda i: (i, 0, 0)),
            pl.BlockSpec(memory_space=pltpu.SMEM),
        ],
        out_shape=[
            jax.ShapeDtypeStruct((_NBN, 1, _BN), jnp.float32),
            jax.ShapeDtypeStruct((_NBN, 1, _BN), jnp.float32),
            jax.ShapeDtypeStruct((1, 1), jnp.float32),
        ],
    )(nf, part4, wnc)


def _tce_body(ef_ref, eas_ref, wec_ref, maxp_ref, p_ref, out_ref):
    ef = ef_ref[...].astype(jnp.bfloat16)
    wet = jnp.transpose(wec_ref[...]).astype(jnp.bfloat16)
    base = lax.dot_general(wet, ef, (((1,), (1,)), ((), ())),
                           preferred_element_type=jnp.float32)
    easmax = jnp.max(maxp_ref[...])
    ein = eas_ref[0] / (easmax + 1e-10)
    ein16 = ein.astype(jnp.bfloat16).astype(jnp.float32)
    w16 = p_ref[0].astype(jnp.bfloat16).astype(jnp.float32)
    z = base + ein16 * w16 + p_ref[1]
    out_ref[0] = jax.nn.sigmoid(z)


def _tce(ef, eas3, wec, maxp2, p):
    return pl.pallas_call(
        _tce_body,
        grid=(_NBE,),
        in_specs=[
            pl.BlockSpec((_BE, 16), lambda i: (i, 0)),
            pl.BlockSpec((1, 1, _BE), lambda i: (i, 0, 0)),
            pl.BlockSpec((16, 1), lambda i: (0, 0)),
            pl.BlockSpec((1, _NW * 16), lambda i: (0, 0)),
            pl.BlockSpec(memory_space=pltpu.SMEM),
        ],
        out_specs=pl.BlockSpec((1, 1, _BE), lambda i: (i, 0, 0)),
        out_shape=jax.ShapeDtypeStruct((_NBE, 1, _BE), jnp.float32),
    )(ef, eas3, wec, maxp2, p)


def _tcn2_body(base_ref, agg_ref, mx_ref, p_ref, out_ref):
    aggmax = mx_ref[0, 0]
    aggn = agg_ref[0] / (aggmax + 1e-10)
    aggn16 = aggn.astype(jnp.bfloat16).astype(jnp.float32)
    w16 = p_ref[0].astype(jnp.bfloat16).astype(jnp.float32)
    z = base_ref[0] + aggn16 * w16 + p_ref[1]
    out_ref[0] = jax.nn.sigmoid(z)


def _tcn2(base3, agg3, mx, p):
    return pl.pallas_call(
        _tcn2_body,
        grid=(_NBN,),
        in_specs=[
            pl.BlockSpec((1, 1, _BN), lambda i: (i, 0, 0)),
            pl.BlockSpec((1, 1, _BN), lambda i: (i, 0, 0)),
            pl.BlockSpec(memory_space=pltpu.SMEM),
            pl.BlockSpec(memory_space=pltpu.SMEM),
        ],
        out_specs=pl.BlockSpec((1, 1, _BN), lambda i: (i, 0, 0)),
        out_shape=jax.ShapeDtypeStruct((_NBN, 1, _BN), jnp.float32),
    )(base3, agg3, mx, p)


def kernel(node_features, edge_features, node_attn_weights, edge_attn_weights,
           Wn, bn, We, be, edge_index):
    naw_flat = node_attn_weights.reshape(-1)
    dst = edge_index[1]
    eas, partial, maxp = _sc_agg(naw_flat, dst)

    part4 = partial.reshape(_NW, _NBN, 1, _BN)
    base3, agg3, mx = _tcn1(node_features, part4, Wn[:128])
    node_scores = _tcn2(base3, agg3, mx,
                        jnp.stack([Wn[128, 0], bn[0]])).reshape(_N)

    edge_scores = _tce(edge_features, eas.reshape(_NBE, 1, _BE), We[:16],
                       maxp.reshape(1, _NW * 16),
                       jnp.stack([We[16, 0], be[0]])).reshape(_E)

    node_k = max(1, int(_N * 0.7))
    edge_k = max(1, int(_E * 0.7))
    _, node_top = lax.top_k(node_scores, node_k)
    node_mask = jnp.zeros((_N,), bool).at[node_top].set(True)
    _, edge_top = lax.top_k(edge_scores, edge_k)
    edge_mask = jnp.zeros((_E,), bool).at[edge_top].set(True)
    return (node_scores, edge_scores, node_mask, edge_mask)
